# baseline (speedup 1.0000x reference)
import jax
import jax.numpy as jnp
from jax import lax
from jax.experimental import pallas as pl
from jax.experimental.pallas import tpu as pltpu
from jax.experimental.pallas import tpu_sc as plsc

_D = 64
_B = 4096
_H = 200
_N = _B * _H
_NW = 32
_PER_W = _N // _NW
_CH = 128
_NCHUNK = _PER_W // _CH
_NBUF = 4
_NGROUP = _NCHUNK // _NBUF


def _body(table, idx, out, idxb, rows, *sems):
    gsem = sems[:_NBUF]
    psem = sems[_NBUF:]
    w = lax.axis_index("s") * 2 + lax.axis_index("c")
    base = w * _PER_W

    pltpu.sync_copy(idx.at[w], idxb)

    for b in range(_NBUF):
        pltpu.async_copy(table.at[idxb.at[b]], rows.at[b], gsem[b])

    @pl.loop(0, _NGROUP - 1)
    def _group(g):
        for b in range(_NBUF):
            j = g * _NBUF + b
            o = out.at[pl.ds(base + j * _CH, _CH), pl.ds(0, _D)]
            pltpu.make_async_copy(
                table.at[idxb.at[j]], rows.at[b], gsem[b]).wait()
            pltpu.async_copy(rows.at[b], o, psem[b])
            pltpu.make_async_copy(rows.at[b], o, psem[b]).wait()
            pltpu.async_copy(table.at[idxb.at[j + _NBUF]], rows.at[b], gsem[b])

    for b in range(_NBUF):
        j = (_NGROUP - 1) * _NBUF + b
        o = out.at[pl.ds(base + j * _CH, _CH), pl.ds(0, _D)]
        pltpu.make_async_copy(table.at[idxb.at[j]], rows.at[b], gsem[b]).wait()
        pltpu.async_copy(rows.at[b], o, psem[b])
    for b in range(_NBUF):
        j = (_NGROUP - 1) * _NBUF + b
        o = out.at[pl.ds(base + j * _CH, _CH), pl.ds(0, _D)]
        pltpu.make_async_copy(rows.at[b], o, psem[b]).wait()


_TW = 1024


def _tp_body(tt_ref, out_ref):
    blk = tt_ref[...]
    blk3 = jnp.reshape(blk, (64, _TW // 2, 2))
    out_ref[:, :64] = jnp.transpose(blk3[:, :, 0], (1, 0))
    out_ref[:, 64:] = jnp.transpose(blk3[:, :, 1], (1, 0))


_tc_pad = pl.pallas_call(
    _tp_body,
    out_shape=jax.ShapeDtypeStruct((500000, 128), jnp.float32),
    grid=(pl.cdiv(1000000, _TW),),
    in_specs=[pl.BlockSpec((64, _TW), lambda i: (0, i))],
    out_specs=pl.BlockSpec((_TW // 2, 128), lambda i: (i, 0)),
)


_mesh = plsc.VectorSubcoreMesh(core_axis_name="c", subcore_axis_name="s")

_gather = pl.kernel(
    _body,
    out_type=jax.ShapeDtypeStruct((_N, 128), jnp.float32),
    mesh=_mesh,
    scratch_types=[
        pltpu.VMEM((_NCHUNK, _CH), jnp.int32),
        pltpu.VMEM((_NBUF, _CH, _D), jnp.float32),
    ] + [pltpu.SemaphoreType.DMA] * (2 * _NBUF),
    compiler_params=pltpu.CompilerParams(use_tc_tiling_on_sc=False),
)


@jax.jit
def kernel(x, embedding):
    idx = x.astype(jnp.int32).reshape(_NW, _NCHUNK, _CH)
    table = _tc_pad(jnp.transpose(embedding)).reshape(1000000, 64)
    out = _gather(table, idx)
    return out.reshape(_B, _H, 128)[:, :, :_D]


# speedup vs baseline: 1.0317x; 1.0317x over previous
import jax
import jax.numpy as jnp
from jax import lax
from jax.experimental import pallas as pl
from jax.experimental.pallas import tpu as pltpu
from jax.experimental.pallas import tpu_sc as plsc

_D = 64
_B = 4096
_H = 200
_N = _B * _H
_NW = 32
_PER_W = _N // _NW
_CH = 128
_NCHUNK = _PER_W // _CH
_NBUF = 4
_NGROUP = _NCHUNK // _NBUF


def _body(table, idx, out, idxb, rows, *sems):
    gsem = sems[:_NBUF]
    psem = sems[_NBUF:]
    w = lax.axis_index("s") * 2 + lax.axis_index("c")
    base = w * _PER_W

    pltpu.sync_copy(idx.at[w], idxb)

    for b in range(_NBUF):
        pltpu.async_copy(table.at[idxb.at[b]], rows.at[b], gsem[b])

    @pl.loop(0, _NGROUP - 1)
    def _group(g):
        for b in range(_NBUF):
            j = g * _NBUF + b
            o = out.at[pl.ds(base + j * _CH, _CH), pl.ds(0, _D)]
            pltpu.make_async_copy(
                table.at[idxb.at[j]], rows.at[b], gsem[b]).wait()
            pltpu.async_copy(rows.at[b], o, psem[b])
            pltpu.make_async_copy(rows.at[b], o, psem[b]).wait()
            pltpu.async_copy(table.at[idxb.at[j + _NBUF]], rows.at[b], gsem[b])

    for b in range(_NBUF):
        j = (_NGROUP - 1) * _NBUF + b
        o = out.at[pl.ds(base + j * _CH, _CH), pl.ds(0, _D)]
        pltpu.make_async_copy(table.at[idxb.at[j]], rows.at[b], gsem[b]).wait()
        pltpu.async_copy(rows.at[b], o, psem[b])
    for b in range(_NBUF):
        j = (_NGROUP - 1) * _NBUF + b
        o = out.at[pl.ds(base + j * _CH, _CH), pl.ds(0, _D)]
        pltpu.make_async_copy(rows.at[b], o, psem[b]).wait()


_TW = 1024


def _tp_body(tt_ref, out_ref):
    blk = tt_ref[...]
    blk3 = jnp.reshape(blk, (64, _TW // 2, 2))
    iot = lax.iota(jnp.int32, 2).astype(jnp.float32)
    ev = lax.dot_general(blk3, 1.0 - iot, (((2,), (0,)), ((), ())))
    od = lax.dot_general(blk3, iot, (((2,), (0,)), ((), ())))
    out_ref[:, :64] = jnp.transpose(ev, (1, 0))
    out_ref[:, 64:] = jnp.transpose(od, (1, 0))


_tc_pad = pl.pallas_call(
    _tp_body,
    out_shape=jax.ShapeDtypeStruct((500000, 128), jnp.float32),
    grid=(pl.cdiv(1000000, _TW),),
    in_specs=[pl.BlockSpec((64, _TW), lambda i: (0, i))],
    out_specs=pl.BlockSpec((_TW // 2, 128), lambda i: (i, 0)),
)


_mesh = plsc.VectorSubcoreMesh(core_axis_name="c", subcore_axis_name="s")

_gather = pl.kernel(
    _body,
    out_type=jax.ShapeDtypeStruct((_N, 128), jnp.float32),
    mesh=_mesh,
    scratch_types=[
        pltpu.VMEM((_NCHUNK, _CH), jnp.int32),
        pltpu.VMEM((_NBUF, _CH, _D), jnp.float32),
    ] + [pltpu.SemaphoreType.DMA] * (2 * _NBUF),
    compiler_params=pltpu.CompilerParams(use_tc_tiling_on_sc=False),
)


@jax.jit
def kernel(x, embedding):
    idx = x.astype(jnp.int32).reshape(_NW, _NCHUNK, _CH)
    table = _tc_pad(jnp.transpose(embedding)).reshape(1000000, 64)
    out = _gather(table, idx)
    return out.reshape(_B, _H, 128)[:, :, :_D]


# speedup vs baseline: 9.7390x; 9.4394x over previous
import jax
import jax.numpy as jnp
from jax import lax
from jax.experimental import pallas as pl
from jax.experimental.pallas import tpu as pltpu
from jax.experimental.pallas import tpu_sc as plsc

_D = 64
_B = 4096
_H = 200
_N = _B * _H
_NW = 32
_PER_W = _N // _NW
_CH = 128
_NCHUNK = _PER_W // _CH
_NBUF = 4
_NGROUP = _NCHUNK // _NBUF


def _body(table, idx, out, idxb, rows, *sems):
    gsem = sems[:_NBUF]
    psem = sems[_NBUF:]
    w = lax.axis_index("s") * 2 + lax.axis_index("c")
    base = w * _PER_W

    pltpu.sync_copy(idx.at[w], idxb)

    for b in range(_NBUF):
        pltpu.async_copy(table.at[idxb.at[b]], rows.at[b], gsem[b])

    @pl.loop(0, _NGROUP - 1)
    def _group(g):
        for b in range(_NBUF):
            j = g * _NBUF + b
            o = out.at[pl.ds(base + j * _CH, _CH), pl.ds(0, _D)]
            pltpu.make_async_copy(
                table.at[idxb.at[j]], rows.at[b], gsem[b]).wait()
            pltpu.async_copy(rows.at[b], o, psem[b])
            pltpu.make_async_copy(rows.at[b], o, psem[b]).wait()
            pltpu.async_copy(table.at[idxb.at[j + _NBUF]], rows.at[b], gsem[b])

    for b in range(_NBUF):
        j = (_NGROUP - 1) * _NBUF + b
        o = out.at[pl.ds(base + j * _CH, _CH), pl.ds(0, _D)]
        pltpu.make_async_copy(table.at[idxb.at[j]], rows.at[b], gsem[b]).wait()
        pltpu.async_copy(rows.at[b], o, psem[b])
    for b in range(_NBUF):
        j = (_NGROUP - 1) * _NBUF + b
        o = out.at[pl.ds(base + j * _CH, _CH), pl.ds(0, _D)]
        pltpu.make_async_copy(rows.at[b], o, psem[b]).wait()


_TW = 1024


def _tp_body(tt_ref, out_ref):
    blk = tt_ref[...]
    i0 = lax.broadcasted_iota(jnp.int32, (_TW, _TW // 2), 0)
    i1 = lax.broadcasted_iota(jnp.int32, (_TW, _TW // 2), 1)
    se = (i0 == 2 * i1).astype(jnp.float32)
    so = (i0 == 2 * i1 + 1).astype(jnp.float32)
    out_ref[:, :64] = lax.dot_general(
        se, blk, (((0,), (1,)), ((), ())),
        preferred_element_type=jnp.float32)
    out_ref[:, 64:] = lax.dot_general(
        so, blk, (((0,), (1,)), ((), ())),
        preferred_element_type=jnp.float32)


_tc_pad = pl.pallas_call(
    _tp_body,
    out_shape=jax.ShapeDtypeStruct((500000, 128), jnp.float32),
    grid=(pl.cdiv(1000000, _TW),),
    in_specs=[pl.BlockSpec((64, _TW), lambda i: (0, i))],
    out_specs=pl.BlockSpec((_TW // 2, 128), lambda i: (i, 0)),
)


_mesh = plsc.VectorSubcoreMesh(core_axis_name="c", subcore_axis_name="s")

_gather = pl.kernel(
    _body,
    out_type=jax.ShapeDtypeStruct((_N, 128), jnp.float32),
    mesh=_mesh,
    scratch_types=[
        pltpu.VMEM((_NCHUNK, _CH), jnp.int32),
        pltpu.VMEM((_NBUF, _CH, _D), jnp.float32),
    ] + [pltpu.SemaphoreType.DMA] * (2 * _NBUF),
    compiler_params=pltpu.CompilerParams(use_tc_tiling_on_sc=False),
)


@jax.jit
def kernel(x, embedding):
    idx = x.astype(jnp.int32).reshape(_NW, _NCHUNK, _CH)
    table = _tc_pad(jnp.transpose(embedding)).reshape(1000000, 64)
    out = _gather(table, idx)
    return out.reshape(_B, _H, 128)[:, :, :_D]


# speedup vs baseline: 9.8234x; 1.0087x over previous
import jax
import jax.numpy as jnp
from jax import lax
from jax.experimental import pallas as pl
from jax.experimental.pallas import tpu as pltpu
from jax.experimental.pallas import tpu_sc as plsc

_D = 64
_B = 4096
_H = 200
_N = _B * _H
_NW = 32
_PER_W = _N // _NW
_CH = 128
_NCHUNK = _PER_W // _CH
_NBUF = 4
_NGROUP = _NCHUNK // _NBUF


def _body(table, idx, out, idxb, rows, *sems):
    gsem = sems[:_NBUF]
    psem = sems[_NBUF:]
    w = lax.axis_index("s") * 2 + lax.axis_index("c")
    base = w * _PER_W

    pltpu.sync_copy(idx.at[w], idxb)

    for b in range(_NBUF):
        pltpu.async_copy(table.at[idxb.at[b]], rows.at[b], gsem[b])

    @pl.loop(0, _NGROUP - 1)
    def _group(g):
        for b in range(_NBUF):
            j = g * _NBUF + b
            o = out.at[pl.ds(base + j * _CH, _CH), pl.ds(0, _D)]
            pltpu.make_async_copy(
                table.at[idxb.at[j]], rows.at[b], gsem[b]).wait()
            pltpu.async_copy(rows.at[b], o, psem[b])
            pltpu.make_async_copy(rows.at[b], o, psem[b]).wait()
            pltpu.async_copy(table.at[idxb.at[j + _NBUF]], rows.at[b], gsem[b])

    for b in range(_NBUF):
        j = (_NGROUP - 1) * _NBUF + b
        o = out.at[pl.ds(base + j * _CH, _CH), pl.ds(0, _D)]
        pltpu.make_async_copy(table.at[idxb.at[j]], rows.at[b], gsem[b]).wait()
        pltpu.async_copy(rows.at[b], o, psem[b])
    for b in range(_NBUF):
        j = (_NGROUP - 1) * _NBUF + b
        o = out.at[pl.ds(base + j * _CH, _CH), pl.ds(0, _D)]
        pltpu.make_async_copy(rows.at[b], o, psem[b]).wait()


_TW = 1024


def _tp_body(tt_ref, se_ref, so_ref, out_ref):
    blk = tt_ref[...]
    se = se_ref[...]
    so = so_ref[...]
    out_ref[:, :64] = lax.dot_general(
        se, blk, (((0,), (1,)), ((), ())),
        preferred_element_type=jnp.float32)
    out_ref[:, 64:] = lax.dot_general(
        so, blk, (((0,), (1,)), ((), ())),
        preferred_element_type=jnp.float32)


_tc_pad = pl.pallas_call(
    _tp_body,
    out_shape=jax.ShapeDtypeStruct((500000, 128), jnp.float32),
    grid=(pl.cdiv(1000000, _TW),),
    in_specs=[pl.BlockSpec((64, _TW), lambda i: (0, i)),
              pl.BlockSpec((_TW, _TW // 2), lambda i: (0, 0)),
              pl.BlockSpec((_TW, _TW // 2), lambda i: (0, 0))],
    out_specs=pl.BlockSpec((_TW // 2, 128), lambda i: (i, 0)),
)


_mesh = plsc.VectorSubcoreMesh(core_axis_name="c", subcore_axis_name="s")

_gather = pl.kernel(
    _body,
    out_type=jax.ShapeDtypeStruct((_N, 128), jnp.float32),
    mesh=_mesh,
    scratch_types=[
        pltpu.VMEM((_NCHUNK, _CH), jnp.int32),
        pltpu.VMEM((_NBUF, _CH, _D), jnp.float32),
    ] + [pltpu.SemaphoreType.DMA] * (2 * _NBUF),
    compiler_params=pltpu.CompilerParams(use_tc_tiling_on_sc=False),
)


@jax.jit
def kernel(x, embedding):
    idx = x.astype(jnp.int32).reshape(_NW, _NCHUNK, _CH)
    i0 = lax.broadcasted_iota(jnp.int32, (_TW, _TW // 2), 0)
    i1 = lax.broadcasted_iota(jnp.int32, (_TW, _TW // 2), 1)
    se = (i0 == 2 * i1).astype(jnp.float32)
    so = (i0 == 2 * i1 + 1).astype(jnp.float32)
    table = _tc_pad(jnp.transpose(embedding), se, so).reshape(1000000, 64)
    out = _gather(table, idx)
    return out.reshape(_B, _H, 128)[:, :, :_D]


# speedup vs baseline: 14.3013x; 1.4558x over previous
import jax
import jax.numpy as jnp
from jax import lax
from jax.experimental import pallas as pl
from jax.experimental.pallas import tpu as pltpu
from jax.experimental.pallas import tpu_sc as plsc

_D = 64
_B = 4096
_H = 200
_N = _B * _H
_NW = 32
_PER_W = _N // _NW
_CH = 128
_NCHUNK = _PER_W // _CH
_NBUF = 4
_NGROUP = _NCHUNK // _NBUF


def _body(table, idx, out, idxb, rows, *sems):
    gsem = sems[:_NBUF]
    psem = sems[_NBUF:]
    w = lax.axis_index("s") * 2 + lax.axis_index("c")
    base = w * _PER_W

    pltpu.sync_copy(idx.at[w], idxb)

    for b in range(_NBUF):
        pltpu.async_copy(table.at[idxb.at[b]], rows.at[b], gsem[b])

    @pl.loop(0, _NGROUP - 1)
    def _group(g):
        for b in range(_NBUF):
            j = g * _NBUF + b
            o = out.at[pl.ds(base + j * _CH, _CH)]
            pltpu.make_async_copy(
                table.at[idxb.at[j]], rows.at[b], gsem[b]).wait()
            pltpu.async_copy(rows.at[b], o, psem[b])
            pltpu.make_async_copy(rows.at[b], o, psem[b]).wait()
            pltpu.async_copy(table.at[idxb.at[j + _NBUF]], rows.at[b], gsem[b])

    for b in range(_NBUF):
        j = (_NGROUP - 1) * _NBUF + b
        o = out.at[pl.ds(base + j * _CH, _CH)]
        pltpu.make_async_copy(table.at[idxb.at[j]], rows.at[b], gsem[b]).wait()
        pltpu.async_copy(rows.at[b], o, psem[b])
    for b in range(_NBUF):
        j = (_NGROUP - 1) * _NBUF + b
        o = out.at[pl.ds(base + j * _CH, _CH)]
        pltpu.make_async_copy(rows.at[b], o, psem[b]).wait()


_TW = 2048


def _tp_body(tt_ref, out_ref):
    blk = tt_ref[...]
    t = jnp.transpose(blk, (1, 0))
    out_ref[:, :64] = t


_tc_pad = pl.pallas_call(
    _tp_body,
    out_shape=jax.ShapeDtypeStruct((1000000, 128), jnp.float32),
    grid=(pl.cdiv(1000000, _TW),),
    in_specs=[pl.BlockSpec((64, _TW), lambda i: (0, i))],
    out_specs=pl.BlockSpec((_TW, 128), lambda i: (i, 0)),
)


_mesh = plsc.VectorSubcoreMesh(core_axis_name="c", subcore_axis_name="s")

_gather = pl.kernel(
    _body,
    out_type=jax.ShapeDtypeStruct((_N, 128), jnp.float32),
    mesh=_mesh,
    scratch_types=[
        pltpu.VMEM((_NCHUNK, _CH), jnp.int32),
        pltpu.VMEM((_NBUF, _CH, 128), jnp.float32),
    ] + [pltpu.SemaphoreType.DMA] * (2 * _NBUF),
)


@jax.jit
def kernel(x, embedding):
    idx = x.astype(jnp.int32).reshape(_NW, _NCHUNK, _CH)
    table = _tc_pad(jnp.transpose(embedding))
    out = _gather(table, idx)
    return out.reshape(_B, _H, 128)[:, :, :_D]
